# baseline (speedup 1.0000x reference)
import functools
import math

import jax
import jax.numpy as jnp
from jax import lax
from jax.experimental import pallas as pl
from jax.experimental.pallas import tpu as pltpu
from jax.experimental.pallas import tpu_sc as plsc

VOCAB = 100000
D = 128
B = 4096
H = 20
NC, NS = 2, 16
NW = NC * NS
FLAT = B * H
PER_W = FLAT // NW
CHUNK = 128
NCH = PER_W // CHUNK
NBUF = 3
SCALE = float(math.sqrt(float(D)))

_mesh = plsc.VectorSubcoreMesh(core_axis_name="c", subcore_axis_name="s")


@functools.partial(
    pl.kernel,
    out_type=jax.ShapeDtypeStruct((FLAT, D), jnp.float32),
    mesh=_mesh,
    scratch_types=[
        pltpu.VMEM((NCH + 4, CHUNK), jnp.int32),
        *[pltpu.VMEM((CHUNK, D), jnp.float32) for _ in range(NBUF)],
        *[pltpu.SemaphoreType.DMA for _ in range(2 * NBUF)],
    ],
)
def _embed_gather(idx_hbm, table_hbm, out_hbm, idx_v, *bufs_and_sems):
    bufs = bufs_and_sems[:NBUF]
    gsems = bufs_and_sems[NBUF:2 * NBUF]
    ssems = bufs_and_sems[2 * NBUF:]
    wid = lax.axis_index("s") * NC + lax.axis_index("c")
    base = wid * PER_W

    start = wid * NCH
    aligned = pl.multiple_of((start // 8) * 8, 8)
    off = start - aligned
    pltpu.sync_copy(idx_hbm.at[pl.ds(aligned, NCH + 4)], idx_v)

    pltpu.async_copy(table_hbm.at[idx_v.at[off]], bufs[0], gsems[0])
    pltpu.async_copy(table_hbm.at[idx_v.at[off + 1]], bufs[1], gsems[1])

    for j in range(NCH):
        b = j % NBUF
        pltpu.make_async_copy(table_hbm.at[idx_v.at[off + j]], bufs[b], gsems[b]).wait()
        if j + 2 < NCH:
            nb = (j + 2) % NBUF
            if j >= 1:
                pltpu.make_async_copy(
                    bufs[nb], out_hbm.at[pl.ds(base + (j - 1) * CHUNK, CHUNK)],
                    ssems[nb],
                ).wait()
            pltpu.async_copy(table_hbm.at[idx_v.at[off + j + 2]], bufs[nb], gsems[nb])
        pltpu.async_copy(bufs[b], out_hbm.at[pl.ds(base + j * CHUNK, CHUNK)],
                         ssems[b])

    for j in range(NCH - NBUF, NCH):
        b = j % NBUF
        pltpu.make_async_copy(
            bufs[b], out_hbm.at[pl.ds(base + j * CHUNK, CHUNK)], ssems[b]
        ).wait()


ROWS_PER_BLK = 32
GRID = B // ROWS_PER_BLK


def _scale_reshape_body(flat_ref, out_ref):
    for i in range(ROWS_PER_BLK):
        out_ref[i] = flat_ref[pl.ds(i * H, H), :] * SCALE


_scale_reshape = pl.pallas_call(
    _scale_reshape_body,
    grid=(GRID,),
    in_specs=[pl.BlockSpec((ROWS_PER_BLK * H, D), lambda i: (i, 0))],
    out_specs=pl.BlockSpec((ROWS_PER_BLK, H, D), lambda i: (i, 0, 0)),
    out_shape=jax.ShapeDtypeStruct((B, H, D), jnp.float32),
    compiler_params=pltpu.CompilerParams(
        dimension_semantics=("arbitrary",),
    ),
)


def kernel(x, input_embedding_table):
    idx = x.astype(jnp.int32).reshape(NW * NCH, CHUNK)
    flat = _embed_gather(idx, input_embedding_table)
    return _scale_reshape(flat)
